# baseline (speedup 1.0000x reference)
import jax
import jax.numpy as jnp
from jax.experimental import pallas as pl
from jax.experimental.pallas import tpu as pltpu

_GW = 128


def _body(xb_ref, btlA_ref, btlB_ref, btA_ref, btB_ref, W1_ref, b1_ref,
          W2r_ref, b2_ref, lo_ref, ov_ref, out_ref,
          numA, denA, numB, denB, m_ref):
    i = pl.program_id(0)
    nb = pl.num_programs(0)
    R = xb_ref.shape[0]
    Rh = R // 2
    G = out_ref.shape[0]

    @pl.when(i == 0)
    def _init():
        numA[...] = jnp.zeros_like(numA)
        denA[...] = jnp.zeros_like(denA)
        numB[...] = jnp.zeros_like(numB)
        denB[...] = jnp.zeros_like(denB)
        m_ref[0, 0] = -jnp.inf
        m_ref[0, 1] = -jnp.inf

    w1 = W1_ref[...].astype(jnp.bfloat16)
    ones16 = jnp.ones((Rh, 128), jnp.bfloat16)
    lo = pl.multiple_of(lo_ref[i], 8)

    halves = ((btlA_ref, btA_ref, numA, denA),
              (btlB_ref, btB_ref, numB, denB))
    xh16s, s16s = [], []

    for k in range(2):
        xh16 = xb_ref[pl.ds(k * Rh, Rh), :].astype(jnp.bfloat16)
        h = jnp.dot(xh16, w1, preferred_element_type=jnp.float32)
        h16 = jnp.maximum(h + b1_ref[...], 0.0).astype(jnp.bfloat16)
        w_row = jax.lax.dot_general(
            W2r_ref[...], h16, (((1,), (1,)), ((), ())),
            preferred_element_type=jnp.float32) + b2_ref[0, 0]

        m_old = m_ref[0, k]
        m_new = jnp.maximum(m_old, jnp.max(w_row))
        m_ref[0, k] = m_new
        scale = jnp.exp(m_old - m_new)
        num_acc, den_acc = halves[k][2], halves[k][3]

        @pl.when(scale < 1.0)
        def _rescale(num_acc=num_acc, den_acc=den_acc, scale=scale):
            num_acc[...] = num_acc[...] * scale
            den_acc[...] = den_acc[...] * scale

        xh16s.append(xh16)
        s16s.append(jnp.exp(w_row - m_new).astype(jnp.bfloat16))

    for k in range(2):
        btl_ref, bt_ref, num_acc, den_acc = halves[k]
        xh16, s16 = xh16s[k], s16s[k]

        @pl.when(ov_ref[i] == 0)
        def _narrow(btl_ref=btl_ref, num_acc=num_acc, den_acc=den_acc,
                    s16=s16, xh16=xh16):
            gid = jax.lax.broadcasted_iota(jnp.int16, (_GW, Rh), 0)
            St = jnp.where(btl_ref[0] == gid, s16, jnp.bfloat16(0.0))
            num_acc[pl.ds(lo, _GW), :] += jnp.dot(
                St, xh16, preferred_element_type=jnp.float32)
            den_acc[pl.ds(lo, _GW), :] += jnp.dot(
                St, ones16, preferred_element_type=jnp.float32)

        @pl.when(ov_ref[i] != 0)
        def _full(bt_ref=bt_ref, num_acc=num_acc, den_acc=den_acc,
                  s16=s16, xh16=xh16):
            gid = jax.lax.broadcasted_iota(jnp.int16, (G, Rh), 0)
            St = jnp.where(bt_ref[0] == gid, s16, jnp.bfloat16(0.0))
            num_acc[:G, :] += jnp.dot(St, xh16,
                                      preferred_element_type=jnp.float32)
            den_acc[:G, :] += jnp.dot(St, ones16,
                                      preferred_element_type=jnp.float32)

    @pl.when(i == nb - 1)
    def _finish():
        mA = m_ref[0, 0]
        mB = m_ref[0, 1]
        M = jnp.maximum(mA, mB)
        fA = jnp.exp(mA - M)
        fB = jnp.exp(mB - M)
        num = numA[:G, :] * fA + numB[:G, :] * fB
        den = denA[:G, 0:1] * fA + denB[:G, 0:1] * fB
        out_ref[...] = num / (den + 1e-6)


def kernel(x, W1, b1, W2, b2, batch):
    N, D = x.shape
    H = W1.shape[1]
    G = 512
    R = 10000
    if N % R != 0:
        R = next(r for r in (10000, 5000, 4000, 2000, 1000, 500, 200, 100,
                             50, 20, 10, 8, N) if N % r == 0)
    NB = N // R
    Rh = R // 2

    batch32 = batch.astype(jnp.int32)
    lo8 = (batch32[::R] // 8) * 8
    over = (batch32[R - 1::R] - lo8 >= _GW).astype(jnp.int32)
    bt2 = batch32.astype(jnp.int16).reshape(NB * 2, 1, Rh)
    btl2 = (batch32 - jnp.repeat(lo8, R)).astype(jnp.int16).reshape(
        NB * 2, 1, Rh)
    b1r = b1.reshape(1, H).astype(jnp.float32)
    W2r = W2.reshape(1, H).astype(jnp.bfloat16)
    b2r = b2.reshape(1, 1).astype(jnp.float32)

    return pl.pallas_call(
        _body,
        grid=(NB,),
        in_specs=[
            pl.BlockSpec((R, D), lambda i: (i, 0)),
            pl.BlockSpec((1, 1, Rh), lambda i: (2 * i, 0, 0)),
            pl.BlockSpec((1, 1, Rh), lambda i: (2 * i + 1, 0, 0)),
            pl.BlockSpec((1, 1, Rh), lambda i: (2 * i, 0, 0)),
            pl.BlockSpec((1, 1, Rh), lambda i: (2 * i + 1, 0, 0)),
            pl.BlockSpec((D, H), lambda i: (0, 0)),
            pl.BlockSpec((1, H), lambda i: (0, 0)),
            pl.BlockSpec((1, H), lambda i: (0, 0)),
            pl.BlockSpec(memory_space=pltpu.SMEM),
            pl.BlockSpec(memory_space=pltpu.SMEM),
            pl.BlockSpec(memory_space=pltpu.SMEM),
        ],
        out_specs=pl.BlockSpec((G, D), lambda i: (0, 0)),
        out_shape=jax.ShapeDtypeStruct((G, D), jnp.float32),
        scratch_shapes=[
            pltpu.VMEM((G + _GW, D), jnp.float32),
            pltpu.VMEM((G + _GW, 128), jnp.float32),
            pltpu.VMEM((G + _GW, D), jnp.float32),
            pltpu.VMEM((G + _GW, 128), jnp.float32),
            pltpu.SMEM((1, 2), jnp.float32),
        ],
        compiler_params=pltpu.CompilerParams(
            dimension_semantics=("arbitrary",)),
    )(x, btl2, btl2, bt2, bt2, W1, b1r, W2r, b2r, lo8, over)


# speedup vs baseline: 1.2750x; 1.2750x over previous
import jax
import jax.numpy as jnp
from jax.experimental import pallas as pl
from jax.experimental.pallas import tpu as pltpu

_GW = 128


def _body(xb_ref, bt_ref, btl_ref, W1_ref, b1_ref, W2r_ref, b2_ref,
          lo_ref, ov_ref, out_ref, num_acc, den_acc, m_ref):
    i = pl.program_id(0)
    nb = pl.num_programs(0)
    R = xb_ref.shape[0]
    G = out_ref.shape[0]

    @pl.when(i == 0)
    def _init():
        num_acc[...] = jnp.zeros_like(num_acc)
        den_acc[...] = jnp.zeros_like(den_acc)
        m_ref[0, 0] = -jnp.inf

    xb16 = xb_ref[...].astype(jnp.bfloat16)
    w1 = W1_ref[...].astype(jnp.bfloat16)
    h = jnp.dot(xb16, w1, preferred_element_type=jnp.float32)
    h16 = jnp.maximum(h + b1_ref[...], 0.0).astype(jnp.bfloat16)
    w_row = jax.lax.dot_general(
        W2r_ref[...], h16, (((1,), (1,)), ((), ())),
        preferred_element_type=jnp.float32) + b2_ref[0, 0]

    m_old = m_ref[0, 0]
    m_new = jnp.maximum(m_old, jnp.max(w_row))
    m_ref[0, 0] = m_new
    scale = jnp.exp(m_old - m_new)

    @pl.when(scale < 1.0)
    def _rescale():
        num_acc[...] = num_acc[...] * scale
        den_acc[...] = den_acc[...] * scale

    s16 = jnp.exp(w_row - m_new).astype(jnp.bfloat16)
    ones16 = jnp.ones((R, 128), jnp.bfloat16)
    lo = pl.multiple_of(lo_ref[i], 8)

    @pl.when(ov_ref[i] == 0)
    def _narrow():
        gid = jax.lax.broadcasted_iota(jnp.int16, (_GW, R), 0)
        St = jnp.where(btl_ref[0] == gid, s16, jnp.bfloat16(0.0))
        num_acc[pl.ds(lo, _GW), :] += jnp.dot(
            St, xb16, preferred_element_type=jnp.float32)
        den_acc[pl.ds(lo, _GW), :] += jnp.dot(
            St, ones16, preferred_element_type=jnp.float32)

    @pl.when(ov_ref[i] != 0)
    def _full():
        gid = jax.lax.broadcasted_iota(jnp.int16, (G, R), 0)
        St = jnp.where(bt_ref[0] == gid, s16, jnp.bfloat16(0.0))
        num_acc[:G, :] += jnp.dot(St, xb16,
                                  preferred_element_type=jnp.float32)
        den_acc[:G, :] += jnp.dot(St, ones16,
                                  preferred_element_type=jnp.float32)

    @pl.when(i == nb - 1)
    def _finish():
        out_ref[...] = num_acc[:G, :] / (den_acc[:G, 0:1] + 1e-6)


def kernel(x, W1, b1, W2, b2, batch):
    N, D = x.shape
    H = W1.shape[1]
    G = 512
    R = 10000
    if N % R != 0:
        R = next(r for r in (10000, 5000, 4000, 2000, 1000, 500, 200, 100,
                             50, 20, 10, 8, N) if N % r == 0)
    NB = N // R

    batch32 = batch.astype(jnp.int32)
    lo8 = (batch32[::R] // 8) * 8
    over = (batch32[R - 1::R] - lo8 >= _GW).astype(jnp.int32)
    bt3 = batch32.astype(jnp.int16).reshape(NB, 1, R)
    btl3 = (batch32 - jnp.repeat(lo8, R)).astype(jnp.int16).reshape(NB, 1, R)
    b1r = b1.reshape(1, H).astype(jnp.float32)
    W2r = W2.reshape(1, H).astype(jnp.bfloat16)
    b2r = b2.reshape(1, 1).astype(jnp.float32)

    return pl.pallas_call(
        _body,
        grid=(NB,),
        in_specs=[
            pl.BlockSpec((R, D), lambda i: (i, 0)),
            pl.BlockSpec((1, 1, R), lambda i: (i, 0, 0)),
            pl.BlockSpec((1, 1, R), lambda i: (i, 0, 0)),
            pl.BlockSpec((D, H), lambda i: (0, 0)),
            pl.BlockSpec((1, H), lambda i: (0, 0)),
            pl.BlockSpec((1, H), lambda i: (0, 0)),
            pl.BlockSpec(memory_space=pltpu.SMEM),
            pl.BlockSpec(memory_space=pltpu.SMEM),
            pl.BlockSpec(memory_space=pltpu.SMEM),
        ],
        out_specs=pl.BlockSpec((G, D), lambda i: (0, 0)),
        out_shape=jax.ShapeDtypeStruct((G, D), jnp.float32),
        scratch_shapes=[
            pltpu.VMEM((G + _GW, D), jnp.float32),
            pltpu.VMEM((G + _GW, 128), jnp.float32),
            pltpu.SMEM((1, 1), jnp.float32),
        ],
        compiler_params=pltpu.CompilerParams(
            dimension_semantics=("arbitrary",)),
    )(x, bt3, btl3, W1, b1r, W2r, b2r, lo8, over)
